# baseline (speedup 1.0000x reference)
import functools

import jax
import jax.numpy as jnp
from jax import lax
from jax.experimental import pallas as pl
from jax.experimental.pallas import tpu as pltpu
from jax.experimental.pallas import tpu_sc as plsc

_NC = 2
_NS = 16
_NW = _NC * _NS
_L = 16
_K = 16
_CHUNK = 128
_DEPTH = 16


def _copy_body(nrow, ncol, ut_hbm, vt_hbm, ou_hbm, ov_hbm, bufs, tailbuf,
               rd_sem, wr_sem):
    wid = lax.axis_index("s") * _NC + lax.axis_index("c")
    ntc = ncol // 128 + (1 if ncol % 128 else 0)
    nfull = ncol // 128
    per_w = nfull // _NW + (1 if nfull % _NW else 0)
    lo = wid * per_w
    hi = jnp.minimum(lo + per_w, nfull)
    n = hi - lo

    def gstart(g):
        return jnp.minimum(lo + g * _DEPTH, hi - _DEPTH)

    ngrp = (n + _DEPTH - 1) // _DEPTH

    for tbl, dst in ((ut_hbm, ou_hbm), (vt_hbm, ov_hbm)):
        for a in range(nrow // 8):
            pltpu.async_copy(
                tbl.at[pl.ds(8 * a, 8), pl.ds(gstart(0) * 128, 128 * _DEPTH)],
                bufs.at[0], rd_sem)

            def grp(g, p, tbl=tbl, dst=dst, a=a):
                s = gstart(g)
                pltpu.make_async_copy(
                    tbl.at[pl.ds(8 * a, 8), pl.ds(0, 128 * _DEPTH)],
                    bufs.at[p], rd_sem).wait()

                @pl.when(g + 1 < ngrp)
                def _():
                    pltpu.async_copy(
                        tbl.at[pl.ds(8 * a, 8),
                               pl.ds(gstart(g + 1) * 128, 128 * _DEPTH)],
                        bufs.at[1 - p], rd_sem)

                wrs = []
                for k in range(_DEPTH):
                    wrs.append(pltpu.async_copy(
                        bufs.at[p, :, pl.ds(k * 128, 128)],
                        dst.at[a * ntc + s + k], wr_sem))
                for c in wrs:
                    c.wait()
                return 1 - p

            lax.fori_loop(0, ngrp, grp, 0)

    if ncol % 128:
        tail = ncol % 128

        @pl.when(wid == 0)
        def _():
            for tbl, dst in ((ut_hbm, ou_hbm), (vt_hbm, ov_hbm)):
                for a in range(nrow // 8):
                    pltpu.sync_copy(
                        tbl.at[pl.ds(8 * a, 8), pl.ds(nfull * 128, tail)],
                        tailbuf)
                    for r in range(8):
                        for c in range(0, tail, 16):
                            bufs[0, r, pl.ds(c, 16)] = tailbuf[r, pl.ds(c, 16)]
                    pltpu.sync_copy(bufs.at[0, :, pl.ds(0, 128)],
                                    dst.at[a * ntc + nfull])


def _tc_retile_body(x_ref, o_ref):
    x = x_ref[...]
    o_ref[...] = x.reshape(8, 16, 128).swapaxes(0, 1).reshape(1, 16, 8, 128)


def _tc_retile(tbl):
    nrow, ncol = tbl.shape
    ntc = ncol // 128 + (1 if ncol % 128 else 0)
    grid = (nrow // 8, (ntc + 15) // 16)
    return pl.pallas_call(
        _tc_retile_body,
        grid=grid,
        in_specs=[pl.BlockSpec((8, 16 * 128), lambda a, j: (a, j))],
        out_specs=pl.BlockSpec((1, 16, 8, 128), lambda a, j: (a, j, 0, 0)),
        out_shape=jax.ShapeDtypeStruct((nrow // 8, ntc, 8, 128), jnp.float32),
    )(tbl)


def _gather_body(n_item, uid_hbm, iid_hbm, ou_hbm, ov_hbm, out_hbm,
                 uidx_v, iidx_v, base_u, base_v, idx_u, idx_v,
                 u_cols, v_cols, out_v, sem):
    wid = lax.axis_index("s") * _NC + lax.axis_index("c")
    b_per_w = out_v.shape[0]
    base = wid * b_per_w
    ntc = n_item // 128 + (1 if n_item % 128 else 0)
    slab = ntc * 8 * 128

    pltpu.sync_copy(uid_hbm.at[pl.ds(base, b_per_w)], uidx_v)
    pltpu.sync_copy(iid_hbm.at[pl.ds(base, b_per_w)], iidx_v)

    def mkbase(r, carry):
        sl = pl.ds(r * _L, _L)
        u = uidx_v[sl]
        base_u[sl] = ((u >> 7) << 10) + (u & 127)
        v = iidx_v[sl]
        base_v[sl] = ((v >> 7) << 10) + (v & 127)
        return carry

    lax.fori_loop(0, b_per_w // _L, mkbase, 0)

    def mkidx(r, carry):
        sl = pl.ds(r * _L, _L)
        bu = base_u[sl]
        bv = base_v[sl]
        for k in range(_K):
            c = (k // 8) * slab + (k % 8) * 128
            idx_u[k, sl] = bu + c
            idx_v[k, sl] = bv + c
        return carry

    lax.fori_loop(0, b_per_w // _L, mkidx, 0)

    n_chunks = b_per_w // _CHUNK

    def fire_k(k, carry):
        for j in range(n_chunks):
            sl = pl.ds(j * _CHUNK, _CHUNK)
            pltpu.async_copy(
                ou_hbm.at[idx_u.at[k, sl]], u_cols.at[k, sl], sem)
            pltpu.async_copy(
                ov_hbm.at[idx_v.at[k, sl]], v_cols.at[k, sl], sem)
        return carry

    lax.fori_loop(0, _K, fire_k, 0)

    def drain_k(k, carry):
        pltpu.make_async_copy(
            ou_hbm.at[pl.ds(0, b_per_w)], u_cols.at[k], sem).wait()
        pltpu.make_async_copy(
            ou_hbm.at[pl.ds(0, b_per_w)], v_cols.at[k], sem).wait()
        return carry

    lax.fori_loop(0, _K, drain_k, 0)

    def group(g, carry):
        sl = pl.ds(g * _L, _L)
        acc = jnp.zeros((_L,), jnp.float32)
        for k in range(_K):
            acc = acc + u_cols[k, sl] * v_cols[k, sl]
        out_v[sl] = acc
        return carry

    lax.fori_loop(0, b_per_w // _L, group, 0)
    pltpu.sync_copy(out_v, out_hbm.at[pl.ds(base, b_per_w)])


def kernel(uid, iid, user_mat, item_mat):
    batch = uid.shape[0]
    b_per_w = batch // _NW
    n_user, kdim = user_mat.shape
    n_item = item_mat.shape[0]
    ntc_u = n_user // 128 + (1 if n_user % 128 else 0)
    ntc_i = n_item // 128 + (1 if n_item % 128 else 0)
    nt_u = (kdim // 8) * ntc_u
    nt_i = (kdim // 8) * ntc_i

    mesh = plsc.VectorSubcoreMesh(core_axis_name="c", subcore_axis_name="s")

    k1 = pl.kernel(
        functools.partial(_copy_body, kdim, n_user),
        out_type=(
            jax.ShapeDtypeStruct((nt_u, 8, 128), jnp.float32),
            jax.ShapeDtypeStruct((nt_i, 8, 128), jnp.float32),
        ),
        mesh=mesh,
        compiler_params=pltpu.CompilerParams(needs_layout_passes=False),
        scratch_types=[
            pltpu.VMEM((2, 8, 128 * _DEPTH), jnp.float32),
            pltpu.VMEM((8, n_user % 128 if n_user % 128 else 128),
                       jnp.float32),
            pltpu.SemaphoreType.DMA,
            pltpu.SemaphoreType.DMA,
        ],
    )
    ou = _tc_retile(user_mat.T)
    ov = _tc_retile(item_mat.T)
    del k1

    k2 = pl.kernel(
        functools.partial(_gather_body, n_item),
        out_type=jax.ShapeDtypeStruct((batch,), jnp.float32),
        mesh=mesh,
        compiler_params=pltpu.CompilerParams(
            needs_layout_passes=False, use_tc_tiling_on_sc=False),
        scratch_types=[
            pltpu.VMEM((b_per_w,), jnp.int32),
            pltpu.VMEM((b_per_w,), jnp.int32),
            pltpu.VMEM((b_per_w,), jnp.int32),
            pltpu.VMEM((b_per_w,), jnp.int32),
            pltpu.VMEM((_K, b_per_w), jnp.int32),
            pltpu.VMEM((_K, b_per_w), jnp.int32),
            pltpu.VMEM((_K, b_per_w), jnp.float32),
            pltpu.VMEM((_K, b_per_w), jnp.float32),
            pltpu.VMEM((b_per_w,), jnp.float32),
            pltpu.SemaphoreType.DMA,
        ],
    )
    return k2(uid.astype(jnp.int32), iid.astype(jnp.int32),
              ou.reshape(-1), ov.reshape(-1))


# speedup vs baseline: 7.1888x; 7.1888x over previous
import functools

import jax
import jax.numpy as jnp
from jax import lax
from jax.experimental import pallas as pl
from jax.experimental.pallas import tpu as pltpu
from jax.experimental.pallas import tpu_sc as plsc

_NC = 2
_NS = 16
_NW = _NC * _NS
_L = 16
_K = 16
_CHUNK = 128
_DEPTH = 32


def _copy_body(nrow, ncol, ut_hbm, vt_hbm, ou_hbm, ov_hbm, bufs, tailbuf,
               rd_sem, wr_sem):
    wid = lax.axis_index("s") * _NC + lax.axis_index("c")
    ntc = ncol // 128 + (1 if ncol % 128 else 0)
    nfull = ncol // 128
    per_w = nfull // _NW + (1 if nfull % _NW else 0)
    lo = wid * per_w
    hi = jnp.minimum(lo + per_w, nfull)
    n = hi - lo

    def gstart(g):
        return jnp.minimum(lo + g * _DEPTH, hi - _DEPTH)

    ngrp = (n + _DEPTH - 1) // _DEPTH

    for tbl, dst in ((ut_hbm, ou_hbm), (vt_hbm, ov_hbm)):
        for a in range(nrow // 8):
            pltpu.async_copy(
                tbl.at[pl.ds(8 * a, 8), pl.ds(gstart(0) * 128, 128 * _DEPTH)],
                bufs.at[0], rd_sem)

            def grp(g, p, tbl=tbl, dst=dst, a=a):
                s = gstart(g)
                pltpu.make_async_copy(
                    tbl.at[pl.ds(8 * a, 8), pl.ds(0, 128 * _DEPTH)],
                    bufs.at[p], rd_sem).wait()

                @pl.when(g + 1 < ngrp)
                def _():
                    pltpu.async_copy(
                        tbl.at[pl.ds(8 * a, 8),
                               pl.ds(gstart(g + 1) * 128, 128 * _DEPTH)],
                        bufs.at[1 - p], rd_sem)

                wrs = []
                for k in range(_DEPTH):
                    wrs.append(pltpu.async_copy(
                        bufs.at[p, :, pl.ds(k * 128, 128)],
                        dst.at[a * ntc + s + k], wr_sem))
                for c in wrs:
                    c.wait()
                return 1 - p

            lax.fori_loop(0, ngrp, grp, 0)

    if ncol % 128:
        tail = ncol % 128

        @pl.when(wid == 0)
        def _():
            for tbl, dst in ((ut_hbm, ou_hbm), (vt_hbm, ov_hbm)):
                for a in range(nrow // 8):
                    pltpu.sync_copy(
                        tbl.at[pl.ds(8 * a, 8), pl.ds(nfull * 128, tail)],
                        tailbuf)
                    for r in range(8):
                        for c in range(0, tail, 16):
                            bufs[0, r, pl.ds(c, 16)] = tailbuf[r, pl.ds(c, 16)]
                    pltpu.sync_copy(bufs.at[0, :, pl.ds(0, 128)],
                                    dst.at[a * ntc + nfull])


def _gather_body(n_item, uid_hbm, iid_hbm, ou_hbm, ov_hbm, out_hbm,
                 uidx_v, iidx_v, base_u, base_v, idx_u, idx_v,
                 u_cols, v_cols, out_v, sem):
    wid = lax.axis_index("s") * _NC + lax.axis_index("c")
    b_per_w = out_v.shape[0]
    base = wid * b_per_w
    ntc = n_item // 128 + (1 if n_item % 128 else 0)
    slab = ntc * 8 * 128

    pltpu.sync_copy(uid_hbm.at[pl.ds(base, b_per_w)], uidx_v)
    pltpu.sync_copy(iid_hbm.at[pl.ds(base, b_per_w)], iidx_v)

    def mkbase(r, carry):
        sl = pl.ds(r * _L, _L)
        u = uidx_v[sl]
        base_u[sl] = ((u >> 7) << 10) + (u & 127)
        v = iidx_v[sl]
        base_v[sl] = ((v >> 7) << 10) + (v & 127)
        return carry

    lax.fori_loop(0, b_per_w // _L, mkbase, 0)

    def mkidx(r, carry):
        sl = pl.ds(r * _L, _L)
        bu = base_u[sl]
        bv = base_v[sl]
        for k in range(_K):
            c = (k // 8) * slab + (k % 8) * 128
            idx_u[k, sl] = bu + c
            idx_v[k, sl] = bv + c
        return carry

    lax.fori_loop(0, b_per_w // _L, mkidx, 0)

    n_chunks = b_per_w // _CHUNK

    def fire_k(k, carry):
        for j in range(n_chunks):
            sl = pl.ds(j * _CHUNK, _CHUNK)
            pltpu.async_copy(
                ou_hbm.at[idx_u.at[k, sl]], u_cols.at[k, sl], sem)
            pltpu.async_copy(
                ov_hbm.at[idx_v.at[k, sl]], v_cols.at[k, sl], sem)
        return carry

    lax.fori_loop(0, _K, fire_k, 0)

    def drain_k(k, carry):
        pltpu.make_async_copy(
            ou_hbm.at[pl.ds(0, b_per_w)], u_cols.at[k], sem).wait()
        pltpu.make_async_copy(
            ou_hbm.at[pl.ds(0, b_per_w)], v_cols.at[k], sem).wait()
        return carry

    lax.fori_loop(0, _K, drain_k, 0)

    def group(g, carry):
        sl = pl.ds(g * _L, _L)
        acc = jnp.zeros((_L,), jnp.float32)
        for k in range(_K):
            acc = acc + u_cols[k, sl] * v_cols[k, sl]
        out_v[sl] = acc
        return carry

    lax.fori_loop(0, b_per_w // _L, group, 0)
    pltpu.sync_copy(out_v, out_hbm.at[pl.ds(base, b_per_w)])


def kernel(uid, iid, user_mat, item_mat):
    batch = uid.shape[0]
    b_per_w = batch // _NW
    n_user, kdim = user_mat.shape
    n_item = item_mat.shape[0]
    ntc_u = n_user // 128 + (1 if n_user % 128 else 0)
    ntc_i = n_item // 128 + (1 if n_item % 128 else 0)
    nt_u = (kdim // 8) * ntc_u
    nt_i = (kdim // 8) * ntc_i

    mesh = plsc.VectorSubcoreMesh(core_axis_name="c", subcore_axis_name="s")

    k1 = pl.kernel(
        functools.partial(_copy_body, kdim, n_user),
        out_type=(
            jax.ShapeDtypeStruct((nt_u, 8, 128), jnp.float32),
            jax.ShapeDtypeStruct((nt_i, 8, 128), jnp.float32),
        ),
        mesh=mesh,
        compiler_params=pltpu.CompilerParams(needs_layout_passes=False),
        scratch_types=[
            pltpu.VMEM((2, 8, 128 * _DEPTH), jnp.float32),
            pltpu.VMEM((8, n_user % 128 if n_user % 128 else 128),
                       jnp.float32),
            pltpu.SemaphoreType.DMA,
            pltpu.SemaphoreType.DMA,
        ],
    )
    ou, ov = k1(user_mat.T, item_mat.T)

    k2 = pl.kernel(
        functools.partial(_gather_body, n_item),
        out_type=jax.ShapeDtypeStruct((batch,), jnp.float32),
        mesh=mesh,
        compiler_params=pltpu.CompilerParams(
            needs_layout_passes=False, use_tc_tiling_on_sc=False),
        scratch_types=[
            pltpu.VMEM((b_per_w,), jnp.int32),
            pltpu.VMEM((b_per_w,), jnp.int32),
            pltpu.VMEM((b_per_w,), jnp.int32),
            pltpu.VMEM((b_per_w,), jnp.int32),
            pltpu.VMEM((_K, b_per_w), jnp.int32),
            pltpu.VMEM((_K, b_per_w), jnp.int32),
            pltpu.VMEM((_K, b_per_w), jnp.float32),
            pltpu.VMEM((_K, b_per_w), jnp.float32),
            pltpu.VMEM((b_per_w,), jnp.float32),
            pltpu.SemaphoreType.DMA,
        ],
    )
    return k2(uid.astype(jnp.int32), iid.astype(jnp.int32),
              ou.reshape(-1), ov.reshape(-1))
